# baseline (speedup 1.0000x reference)
import functools

import jax
import jax.numpy as jnp
from jax import lax
from jax.experimental import pallas as pl
from jax.experimental.pallas import tpu as pltpu
from jax.experimental.pallas import tpu_sc as plsc

MIN_S = 0.05
MAX_S = 0.3
B = 8192
D = 4096
H = D // 4
BR = 128
NBLK = B // BR
MAX_FINITE_BITS = 0x7F7FFFFF

NC = 2
NS = 16
NW = NC * NS
RPW = B // NW
L = 16
SL = D // L
CU = 8


def _mlp_kernel(x_ref, w1_ref, b1_ref, w2_ref, b2_ref, sp_ref, k_ref):
    x = x_ref[...]
    h = jnp.maximum(
        lax.dot_general(x.astype(jnp.bfloat16), w1_ref[...],
                        (((1,), (0,)), ((), ())),
                        preferred_element_type=jnp.float32)
        + b1_ref[...], 0.0)
    hb = h.astype(jnp.bfloat16).astype(jnp.float32)
    z = jnp.sum(hb * w2_ref[...].astype(jnp.float32), axis=1,
                keepdims=True) + b2_ref[...]
    s = jax.nn.sigmoid(z)
    sparsity = MIN_S + (MAX_S - MIN_S) * s
    sp_ref[...] = sparsity
    kf = jnp.round(D * (1.0 - sparsity))
    k_ref[...] = jnp.maximum(1, kf.astype(jnp.int32))


def _sc_kernel(x_hbm, k_hbm, sp_hbm, m_hbm, asp_hbm, l1_hbm,
               k_v, row0_v, row1_v, bits_v, spb0_v, spb1_v, mb0_v, mb1_v,
               asp_v, l1b_v, cand_v, pc_v, off_v, rs0, rs1,
               os0, os1):
    c = lax.axis_index("c")
    s = lax.axis_index("s")
    wid = s * NC + c
    base = wid * RPW
    pltpu.sync_copy(k_hbm.at[pl.ds(base, RPW)], k_v.at[pl.ds(0, RPW)])

    def fetch(g, rv, rs):
        pltpu.make_async_copy(x_hbm.at[base + g], rv, rs).start()

    def process(g, rv, spp, mbp, rs, os, carry):
        l1vec, aspvec = carry
        row = base + g
        pltpu.make_async_copy(x_hbm.at[row], rv, rs).wait()

        kk = k_v[pl.ds(g, L)][0]
        lane = lax.iota(jnp.int32, L)

        def bits_body(j, t):
            for u in range(CU):
                off = (j * CU + u) * L
                xv = rv[pl.ds(off, L)]
                bits_v[pl.ds(off, L)] = (
                    lax.bitcast_convert_type(xv, jnp.int32)
                    & jnp.int32(0x7FFFFFFF))
            return t
        lax.fori_loop(0, SL // CU, bits_body, 0, unroll=False)

        def bsA_body(_, carry):
            lo, hi, nbelow = carry
            mid = lo + ((hi - lo) >> 1)

            def cnt_body(j, acc):
                for u in range(CU):
                    off = (j * CU + u) * L
                    bv = bits_v[pl.ds(off, L)]
                    acc = acc + plsc.all_reduce_population_count(bv <= mid)
                return acc
            acc = lax.fori_loop(0, SL // CU, cnt_body,
                                jnp.zeros((L,), jnp.int32), unroll=False)
            cnt = acc[0]
            pred = cnt >= kk
            return (jnp.where(pred, lo, mid + 1), jnp.where(pred, mid, hi),
                    jnp.where(pred, nbelow, cnt))

        lo, hi, nbelow = lax.fori_loop(
            0, 12, bsA_body,
            (jnp.int32(0), jnp.int32(MAX_FINITE_BITS), jnp.int32(0)))
        kk2 = kk - nbelow

        def pc_body(j, t):
            for u in range(CU):
                jj = j * CU + u
                bv = bits_v[pl.ds(jj * L, L)]
                m = (bv >= lo) & (bv <= hi)
                pc = plsc.all_reduce_population_count(m)
                plsc.store_compressed(pc_v.at[pl.ds(jj, L)], pc,
                                      mask=lane == 0)
            return t
        lax.fori_loop(0, SL // CU, pc_body, 0, unroll=False)

        def scan_body(t, carry):
            v = pc_v[pl.ds(t * L, L)]
            cs = plsc.cumsum(v)
            off_v[pl.ds(t * L, L)] = cs - v + carry
            return carry + cs[L - 1]
        ns = lax.fori_loop(0, SL // L, scan_body, jnp.int32(0), unroll=False)
        nsl = (ns + (L - 1)) >> 4

        def sc_body(j, t):
            for u in range(CU):
                jj = j * CU + u
                off = off_v[pl.ds(jj, L)][0]
                bv = bits_v[pl.ds(jj * L, L)]
                m = (bv >= lo) & (bv <= hi)
                plsc.store_compressed(cand_v.at[pl.ds(off, L)], bv, mask=m)
            return t
        lax.fori_loop(0, SL // CU, sc_body, 0, unroll=False)

        def bsC_body(_, lohi):
            lo, hi = lohi
            mid = lo + ((hi - lo) >> 1)

            def cnt_body(j, acc):
                bv = cand_v[pl.ds(j * L, L)]
                lm = ((j * L + lane) < ns) & (bv <= mid)
                return acc + plsc.all_reduce_population_count(lm)
            acc = lax.fori_loop(0, nsl, cnt_body,
                                jnp.zeros((L,), jnp.int32), unroll=False)
            pred = acc[0] >= kk2
            return (jnp.where(pred, lo, mid + 1), jnp.where(pred, mid, hi))

        thr, _ = lax.fori_loop(0, 19, bsC_body, (lo, hi))

        @pl.when(g >= 2)
        def _():
            pltpu.make_async_copy(spp, sp_hbm.at[row - 2], os).wait()
            pltpu.make_async_copy(mbp, m_hbm.at[row - 2], os).wait()

        def mask_body(j, carry):
            cntv, l1a = carry
            for u in range(CU):
                off = (j * CU + u) * L
                xv = rv[pl.ds(off, L)]
                bv = bits_v[pl.ds(off, L)]
                m = bv > thr
                mf = jnp.where(m, jnp.float32(1.0), jnp.float32(0.0))
                mbp[pl.ds(off, L)] = mf
                spp[pl.ds(off, L)] = jnp.where(m, xv, jnp.float32(0.0))
                cntv = cntv + plsc.all_reduce_population_count(m)
                l1a = l1a + jnp.where(
                    m, lax.bitcast_convert_type(bv, jnp.float32),
                    jnp.float32(0.0))
            return (cntv, l1a)
        cntv, l1a = lax.fori_loop(
            0, SL // CU, mask_body,
            (jnp.zeros((L,), jnp.int32), jnp.zeros((L,), jnp.float32)),
            unroll=False)

        pltpu.make_async_copy(spp, sp_hbm.at[row], os).start()
        pltpu.make_async_copy(mbp, m_hbm.at[row], os).start()

        aspsplat = cntv.astype(jnp.float32) * (1.0 / D)
        lane = lax.iota(jnp.int32, L)
        aspvec = aspvec + jnp.where(lane == (g % L), aspsplat,
                                    jnp.float32(0.0))
        flush = (g % L) == (L - 1)

        @pl.when(flush)
        def _():
            asp_v[pl.ds(g - (L - 1), L)] = aspvec

        aspvec = jnp.where(flush, jnp.zeros((L,), jnp.float32), aspvec)
        return (l1vec + l1a, aspvec)

    fetch(0, row0_v, rs0)

    def pair_body(i, carry):
        g0 = 2 * i

        @pl.when(g0 + 1 < RPW)
        def _():
            fetch(g0 + 1, row1_v, rs1)
        carry = process(g0, row0_v, spb0_v, mb0_v, rs0, os0, carry)

        @pl.when(g0 + 2 < RPW)
        def _():
            fetch(g0 + 2, row0_v, rs0)
        carry = process(g0 + 1, row1_v, spb1_v, mb1_v, rs1, os1, carry)
        return carry

    l1vec, _ = lax.fori_loop(0, RPW // 2, pair_body,
                             (jnp.zeros((L,), jnp.float32),
                              jnp.zeros((L,), jnp.float32)), unroll=False)
    pltpu.make_async_copy(spb0_v, sp_hbm.at[base + RPW - 2], os0).wait()
    pltpu.make_async_copy(mb0_v, m_hbm.at[base + RPW - 2], os0).wait()
    pltpu.make_async_copy(spb1_v, sp_hbm.at[base + RPW - 1], os1).wait()
    pltpu.make_async_copy(mb1_v, m_hbm.at[base + RPW - 1], os1).wait()
    l1b_v[...] = l1vec
    pltpu.sync_copy(l1b_v, l1_hbm.at[pl.ds(wid * L, L)])
    pltpu.sync_copy(asp_v, asp_hbm.at[pl.ds(base, RPW)])


@functools.partial(
    pl.kernel,
    out_type=(
        jax.ShapeDtypeStruct((B, D), jnp.float32),
        jax.ShapeDtypeStruct((B, D), jnp.float32),
        jax.ShapeDtypeStruct((B,), jnp.float32),
        jax.ShapeDtypeStruct((NW * L,), jnp.float32),
    ),
    mesh=plsc.VectorSubcoreMesh(core_axis_name="c", subcore_axis_name="s"),
    compiler_params=pltpu.CompilerParams(needs_layout_passes=False),
    scratch_types=[
        pltpu.VMEM((RPW + L,), jnp.int32),
        pltpu.VMEM((D,), jnp.float32),
        pltpu.VMEM((D,), jnp.float32),
        pltpu.VMEM((D,), jnp.int32),
        pltpu.VMEM((D,), jnp.float32),
        pltpu.VMEM((D,), jnp.float32),
        pltpu.VMEM((D,), jnp.float32),
        pltpu.VMEM((D,), jnp.float32),
        pltpu.VMEM((RPW,), jnp.float32),
        pltpu.VMEM((L,), jnp.float32),
        pltpu.VMEM((D + L,), jnp.int32),
        pltpu.VMEM((SL + L,), jnp.int32),
        pltpu.VMEM((SL + L,), jnp.int32),
        pltpu.SemaphoreType.DMA,
        pltpu.SemaphoreType.DMA,
        pltpu.SemaphoreType.DMA,
        pltpu.SemaphoreType.DMA,
    ],
)
def _sc_call(x_in, k_in, sp_out, m_out, asp_out, l1_out,
             k_v, row0_v, row1_v, bits_v, spb0_v, spb1_v, mb0_v, mb1_v,
             asp_v, l1b_v, cand_v, pc_v, off_v,
             rs0, rs1, os0, os1):
    _sc_kernel(x_in, k_in, sp_out, m_out, asp_out, l1_out,
               k_v, row0_v, row1_v, bits_v, spb0_v, spb1_v, mb0_v, mb1_v,
               asp_v, l1b_v, cand_v, pc_v, off_v,
               rs0, rs1, os0, os1)


@jax.jit
def kernel(x, W1, b1, W2, b2):
    b1r = b1.reshape(1, H)
    w2r = W2.reshape(1, H).astype(jnp.bfloat16)
    b2r = b2.reshape(1, 1)
    W1b = W1.astype(jnp.bfloat16)
    row_spec = pl.BlockSpec((BR, 1), lambda i: (i, 0))
    full = lambda shape: pl.BlockSpec(shape, lambda i: (0,) * len(shape))
    sparsity, kk = pl.pallas_call(
        _mlp_kernel,
        grid=(NBLK,),
        in_specs=[pl.BlockSpec((BR, D), lambda i: (i, 0)), full((D, H)),
                  full((1, H)), full((1, H)), full((1, 1))],
        out_specs=(row_spec, row_spec),
        out_shape=(jax.ShapeDtypeStruct((B, 1), jnp.float32),
                   jax.ShapeDtypeStruct((B, 1), jnp.int32)),
    )(x, W1b, b1r, w2r, b2r)
    sparse_x, mask, asp, l1p = _sc_call(x, kk.reshape(B))
    return sparse_x, mask, sparsity, asp, jnp.sum(l1p) * (1.0 / B)


# speedup vs baseline: 1.4652x; 1.4652x over previous
import functools

import jax
import jax.numpy as jnp
from jax import lax
from jax.experimental import pallas as pl
from jax.experimental.pallas import tpu as pltpu
from jax.experimental.pallas import tpu_sc as plsc

MIN_S = 0.05
MAX_S = 0.3
B = 8192
D = 4096
H = D // 4
BR = 128
NBLK = B // BR
MAX_FINITE_BITS = 0x7F7FFFFF

NC = 2
NS = 16
NW = NC * NS
RPW = B // NW
L = 16
SL = D // L
CU = 8


def _mlp_kernel(x_ref, w1_ref, b1_ref, w2_ref, b2_ref, sp_ref, k_ref):
    x = x_ref[...]
    h = jnp.maximum(
        lax.dot_general(x.astype(jnp.bfloat16), w1_ref[...],
                        (((1,), (0,)), ((), ())),
                        preferred_element_type=jnp.float32)
        + b1_ref[...], 0.0)
    hb = h.astype(jnp.bfloat16).astype(jnp.float32)
    z = jnp.sum(hb * w2_ref[...].astype(jnp.float32), axis=1,
                keepdims=True) + b2_ref[...]
    s = jax.nn.sigmoid(z)
    sparsity = MIN_S + (MAX_S - MIN_S) * s
    sp_ref[...] = sparsity
    kf = jnp.round(D * (1.0 - sparsity))
    k_ref[...] = jnp.maximum(1, kf.astype(jnp.int32))


def _sc_kernel(x_hbm, k_hbm, sp_hbm, m_hbm, asp_hbm, l1_hbm,
               k_v, row0_v, row1_v, bits_v, spb0_v, spb1_v, mb0_v, mb1_v,
               asp_v, l1b_v, rs0, rs1, os0, os1):
    c = lax.axis_index("c")
    s = lax.axis_index("s")
    wid = s * NC + c
    base = wid * RPW
    pltpu.sync_copy(k_hbm.at[pl.ds(base, RPW)], k_v.at[pl.ds(0, RPW)])

    def fetch(g, rv, rs):
        pltpu.make_async_copy(x_hbm.at[base + g], rv, rs).start()

    def process(g, rv, spp, mbp, rs, os, carry):
        l1vec, aspvec = carry
        row = base + g
        pltpu.make_async_copy(x_hbm.at[row], rv, rs).wait()

        kk = k_v[pl.ds(g, L)][0]
        lane = lax.iota(jnp.int32, L)

        def bits_body(j, t):
            for u in range(CU):
                off = (j * CU + u) * L
                xv = rv[pl.ds(off, L)]
                bits_v[pl.ds(off, L)] = (
                    lax.bitcast_convert_type(xv, jnp.int32)
                    & jnp.int32(0x7FFFFFFF))
            return t
        lax.fori_loop(0, SL // CU, bits_body, 0, unroll=False)

        def bs_body(_, lohi):
            lo, hi = lohi
            mid = lo + ((hi - lo) >> 1)

            def cnt_body(j, acc):
                for u in range(CU):
                    off = (j * CU + u) * L
                    bv = bits_v[pl.ds(off, L)]
                    acc = acc + plsc.all_reduce_population_count(bv <= mid)
                return acc
            acc = lax.fori_loop(0, SL // CU, cnt_body,
                                jnp.zeros((L,), jnp.int32), unroll=False)
            pred = acc[0] >= kk
            return (jnp.where(pred, lo, mid + 1), jnp.where(pred, mid, hi))

        thr, _ = lax.fori_loop(0, 31, bs_body,
                               (jnp.int32(0), jnp.int32(MAX_FINITE_BITS)))

        @pl.when(g >= 2)
        def _():
            pltpu.make_async_copy(spp, sp_hbm.at[row - 2], os).wait()
            pltpu.make_async_copy(mbp, m_hbm.at[row - 2], os).wait()

        def mask_body(j, carry):
            cntv, l1a = carry
            for u in range(CU):
                off = (j * CU + u) * L
                xv = rv[pl.ds(off, L)]
                bv = bits_v[pl.ds(off, L)]
                m = bv > thr
                mf = jnp.where(m, jnp.float32(1.0), jnp.float32(0.0))
                mbp[pl.ds(off, L)] = mf
                spp[pl.ds(off, L)] = jnp.where(m, xv, jnp.float32(0.0))
                cntv = cntv + plsc.all_reduce_population_count(m)
                l1a = l1a + jnp.where(
                    m, lax.bitcast_convert_type(bv, jnp.float32),
                    jnp.float32(0.0))
            return (cntv, l1a)
        cntv, l1a = lax.fori_loop(
            0, SL // CU, mask_body,
            (jnp.zeros((L,), jnp.int32), jnp.zeros((L,), jnp.float32)),
            unroll=False)

        pltpu.make_async_copy(spp, sp_hbm.at[row], os).start()
        pltpu.make_async_copy(mbp, m_hbm.at[row], os).start()

        aspsplat = cntv.astype(jnp.float32) * (1.0 / D)
        lane = lax.iota(jnp.int32, L)
        aspvec = aspvec + jnp.where(lane == (g % L), aspsplat,
                                    jnp.float32(0.0))
        flush = (g % L) == (L - 1)

        @pl.when(flush)
        def _():
            asp_v[pl.ds(g - (L - 1), L)] = aspvec

        aspvec = jnp.where(flush, jnp.zeros((L,), jnp.float32), aspvec)
        return (l1vec + l1a, aspvec)

    fetch(0, row0_v, rs0)

    def pair_body(i, carry):
        g0 = 2 * i

        @pl.when(g0 + 1 < RPW)
        def _():
            fetch(g0 + 1, row1_v, rs1)
        carry = process(g0, row0_v, spb0_v, mb0_v, rs0, os0, carry)

        @pl.when(g0 + 2 < RPW)
        def _():
            fetch(g0 + 2, row0_v, rs0)
        carry = process(g0 + 1, row1_v, spb1_v, mb1_v, rs1, os1, carry)
        return carry

    l1vec, _ = lax.fori_loop(0, RPW // 2, pair_body,
                             (jnp.zeros((L,), jnp.float32),
                              jnp.zeros((L,), jnp.float32)), unroll=False)
    pltpu.make_async_copy(spb0_v, sp_hbm.at[base + RPW - 2], os0).wait()
    pltpu.make_async_copy(mb0_v, m_hbm.at[base + RPW - 2], os0).wait()
    pltpu.make_async_copy(spb1_v, sp_hbm.at[base + RPW - 1], os1).wait()
    pltpu.make_async_copy(mb1_v, m_hbm.at[base + RPW - 1], os1).wait()
    l1b_v[...] = l1vec
    pltpu.sync_copy(l1b_v, l1_hbm.at[pl.ds(wid * L, L)])
    pltpu.sync_copy(asp_v, asp_hbm.at[pl.ds(base, RPW)])


@functools.partial(
    pl.kernel,
    out_type=(
        jax.ShapeDtypeStruct((B, D), jnp.float32),
        jax.ShapeDtypeStruct((B, D), jnp.float32),
        jax.ShapeDtypeStruct((B,), jnp.float32),
        jax.ShapeDtypeStruct((NW * L,), jnp.float32),
    ),
    mesh=plsc.VectorSubcoreMesh(core_axis_name="c", subcore_axis_name="s"),
    compiler_params=pltpu.CompilerParams(needs_layout_passes=False),
    scratch_types=[
        pltpu.VMEM((RPW + L,), jnp.int32),
        pltpu.VMEM((D,), jnp.float32),
        pltpu.VMEM((D,), jnp.float32),
        pltpu.VMEM((D,), jnp.int32),
        pltpu.VMEM((D,), jnp.float32),
        pltpu.VMEM((D,), jnp.float32),
        pltpu.VMEM((D,), jnp.float32),
        pltpu.VMEM((D,), jnp.float32),
        pltpu.VMEM((RPW,), jnp.float32),
        pltpu.VMEM((L,), jnp.float32),
        pltpu.SemaphoreType.DMA,
        pltpu.SemaphoreType.DMA,
        pltpu.SemaphoreType.DMA,
        pltpu.SemaphoreType.DMA,
    ],
)
def _sc_call(x_in, k_in, sp_out, m_out, asp_out, l1_out,
             k_v, row0_v, row1_v, bits_v, spb0_v, spb1_v, mb0_v, mb1_v,
             asp_v, l1b_v, rs0, rs1, os0, os1):
    _sc_kernel(x_in, k_in, sp_out, m_out, asp_out, l1_out,
               k_v, row0_v, row1_v, bits_v, spb0_v, spb1_v, mb0_v, mb1_v,
               asp_v, l1b_v, rs0, rs1, os0, os1)


@jax.jit
def kernel(x, W1, b1, W2, b2):
    b1r = b1.reshape(1, H)
    w2r = W2.reshape(1, H).astype(jnp.bfloat16)
    b2r = b2.reshape(1, 1)
    W1b = W1.astype(jnp.bfloat16)
    row_spec = pl.BlockSpec((BR, 1), lambda i: (i, 0))
    full = lambda shape: pl.BlockSpec(shape, lambda i: (0,) * len(shape))
    sparsity, kk = pl.pallas_call(
        _mlp_kernel,
        grid=(NBLK,),
        in_specs=[pl.BlockSpec((BR, D), lambda i: (i, 0)), full((D, H)),
                  full((1, H)), full((1, H)), full((1, 1))],
        out_specs=(row_spec, row_spec),
        out_shape=(jax.ShapeDtypeStruct((B, 1), jnp.float32),
                   jax.ShapeDtypeStruct((B, 1), jnp.int32)),
    )(x, W1b, b1r, w2r, b2r)
    sparse_x, mask, asp, l1p = _sc_call(x, kk.reshape(B))
    return sparse_x, mask, sparsity, asp, jnp.sum(l1p) * (1.0 / B)
